# baseline (speedup 1.0000x reference)
import functools

import jax
import jax.numpy as jnp
from jax import lax
from jax.experimental import pallas as pl
from jax.experimental.pallas import tpu as pltpu
from jax.experimental.pallas import tpu_sc as plsc

N_NODES = 10000
D = 128
N_EDGES = 320000

NC = 2
NS = 16
NW = NC * NS
EPB = 64
BLKS_PER_ADJ = 5120
E_PAD = BLKS_PER_ADJ * EPB
BPT1 = BLKS_PER_ADJ // NW
BPT2 = 2 * BLKS_PER_ADJ // NW
N_PAD = 10240
RPS = N_PAD // NS
NBUF = 4
CH = 16


def _scale_block(buf, w_ref, row, scale):
    def grp(g, _):
        w16 = w_ref[row, pl.ds(g * 16, 16)] * scale
        for e in range(16):
            wb = w16[e]
            r = g * 16 + e
            for j in range(8):
                sl = pl.ds(16 * j, 16)
                buf[r, sl] = buf[r, sl] * wb
        return 0

    lax.fori_loop(0, EPB // 16, grp, 0, unroll=False)


def _spmm_tile(tbl_hbm, src2d, dst2d, w2d, out_hbm,
               src_r, dst_r, w_r, bufs, acc, isem, gsem, ssem,
               c, s, bpt, scale):
    tile = c * NS + s
    tb0 = tile * bpt
    nch = bpt // CH
    RING = 2 * CH

    d0 = pltpu.async_copy(src2d.at[pl.ds(tb0, CH)],
                          src_r.at[pl.ds(0, CH)], isem)
    d1 = pltpu.async_copy(dst2d.at[pl.ds(tb0, CH)],
                          dst_r.at[pl.ds(0, CH)], isem)
    d2 = pltpu.async_copy(w2d.at[pl.ds(tb0, CH)],
                          w_r.at[pl.ds(0, CH)], isem)

    zeros = jnp.zeros((16,), jnp.float32)

    def zrow(i, _):
        for j in range(8):
            bufs[0][i, pl.ds(16 * j, 16)] = zeros
        return 0

    lax.fori_loop(0, EPB, zrow, 0, unroll=False)
    for k in range(RPS // EPB):
        pltpu.sync_copy(bufs[0], acc.at[pl.ds(s * RPS + k * EPB, EPB)])

    d0.wait()
    d1.wait()
    d2.wait()
    plsc.subcore_barrier()

    def fire_g(i, buf):
        pltpu.async_copy(tbl_hbm.at[src_r.at[i % RING]], buf, gsem)

    def wait_g(buf):
        pltpu.make_async_copy(tbl_hbm.at[src_r.at[0]], buf, gsem).wait()

    def fire_s(i, buf):
        pltpu.async_copy(buf, acc.at[dst_r.at[i % RING]], ssem, add=True)

    def wait_s():
        pltpu.make_async_copy(bufs[0], acc.at[dst_r.at[0]], ssem).wait()

    for i in range(NBUF - 1):
        fire_g(i, bufs[i])

    def rnd(r, _):
        for u in range(NBUF):
            i = r * NBUF + u
            buf = bufs[u]
            wait_g(buf)
            _scale_block(buf, w_r, i % RING, scale)

            @pl.when(i < bpt - (NBUF - 1))
            def _():
                @pl.when(i > 0)
                def _():
                    wait_s()
                fire_g(i + NBUF - 1, bufs[(u + NBUF - 1) % NBUF])

            fire_s(i, buf)
            if u == 1:
                ci = i // CH

                @pl.when(jnp.logical_and(i % CH == 1, ci < nch - 1))
                def _():
                    r0 = tb0 + (ci + 1) * CH
                    rr = ((ci + 1) % 2) * CH
                    pltpu.async_copy(src2d.at[pl.ds(r0, CH)],
                                     src_r.at[pl.ds(rr, CH)], isem)
                    pltpu.async_copy(dst2d.at[pl.ds(r0, CH)],
                                     dst_r.at[pl.ds(rr, CH)], isem)
                    pltpu.async_copy(w2d.at[pl.ds(r0, CH)],
                                     w_r.at[pl.ds(rr, CH)], isem)

                @pl.when(jnp.logical_and(i % CH == CH - NBUF + 1,
                                         ci < nch - 1))
                def _():
                    for rf in (src_r, dst_r, w_r):
                        pltpu.make_async_copy(src2d.at[pl.ds(0, CH)],
                                              rf.at[pl.ds(0, CH)],
                                              isem).wait()
        return 0

    lax.fori_loop(0, bpt // NBUF, rnd, 0, unroll=False)

    for _ in range(NBUF):
        wait_s()
    plsc.subcore_barrier()

    for k in range(RPS // EPB):
        r0 = s * RPS + k * EPB
        pltpu.async_copy(acc.at[pl.ds(r0, EPB)],
                         out_hbm.at[c, pl.ds(r0, EPB)], isem)
    for k in range(RPS // EPB):
        pltpu.make_async_copy(acc.at[pl.ds(0, EPB)],
                              out_hbm.at[0, pl.ds(0, EPB)], isem).wait()


_SPMM_SCRATCH = [
    pltpu.VMEM((2 * CH, EPB), jnp.int32),
    pltpu.VMEM((2 * CH, EPB), jnp.int32),
    pltpu.VMEM((2 * CH, EPB), jnp.float32),
    [pltpu.VMEM((EPB, D), jnp.float32) for _ in range(NBUF)],
    pltpu.VMEM_SHARED((N_PAD, D), jnp.float32),
    pltpu.SemaphoreType.DMA,
    pltpu.SemaphoreType.DMA,
    pltpu.SemaphoreType.DMA,
]

_SC_MESH = plsc.VectorSubcoreMesh(core_axis_name="c", subcore_axis_name="s")


@functools.partial(
    pl.kernel,
    out_type=jax.ShapeDtypeStruct((NC, N_PAD, D), jnp.float32),
    mesh=_SC_MESH,
    scratch_types=_SPMM_SCRATCH,
)
def _sc_spmm_pair(src2d, dst2d, w2d, h_hbm, out_hbm,
                  src_r, dst_r, w_r, bufs, acc, isem, gsem, ssem):
    c = lax.axis_index("c")
    s = lax.axis_index("s")
    _spmm_tile(h_hbm, src2d, dst2d, w2d, out_hbm,
               src_r, dst_r, w_r, bufs, acc, isem, gsem, ssem,
               c, s, BPT2, 0.5)


@functools.partial(
    pl.kernel,
    out_type=jax.ShapeDtypeStruct((NC, N_PAD, D), jnp.float32),
    mesh=_SC_MESH,
    scratch_types=_SPMM_SCRATCH,
)
def _sc_spmm_single(src2d, dst2d, w2d, s1_hbm, out_hbm,
                    src_r, dst_r, w_r, bufs, acc, isem, gsem, ssem):
    c = lax.axis_index("c")
    s = lax.axis_index("s")
    _spmm_tile(s1_hbm, src2d, dst2d, w2d, out_hbm,
               src_r, dst_r, w_r, bufs, acc, isem, gsem, ssem,
               c, s, BPT1, 1.0)


_ROWS_BLK = 1000


def _tc_affine_body(x_ref, w_ref, b_ref, o_ref):
    o_ref[...] = lax.dot_general(
        x_ref[...], w_ref[...],
        (((1,), (1,)), ((), ())),
        preferred_element_type=jnp.float32,
    ) + b_ref[...]


def _tc_affine(x, W, b):
    return pl.pallas_call(
        _tc_affine_body,
        out_shape=jax.ShapeDtypeStruct((N_NODES, D), jnp.float32),
        grid=(N_NODES // _ROWS_BLK,),
        in_specs=[
            pl.BlockSpec((_ROWS_BLK, D), lambda i: (i, 0)),
            pl.BlockSpec((D, D), lambda i: (0, 0)),
            pl.BlockSpec((1, D), lambda i: (0, 0)),
        ],
        out_specs=pl.BlockSpec((_ROWS_BLK, D), lambda i: (i, 0)),
    )(x, W, b.reshape(1, D))


def _tc_sum_pair_body(p_ref, o_ref):
    o_ref[...] = p_ref[0] + p_ref[1]


def _tc_sum_pair(p):
    return pl.pallas_call(
        _tc_sum_pair_body,
        out_shape=jax.ShapeDtypeStruct((N_NODES, D), jnp.float32),
        grid=(N_NODES // _ROWS_BLK,),
        in_specs=[pl.BlockSpec((NC, _ROWS_BLK, D), lambda i: (0, i, 0))],
        out_specs=pl.BlockSpec((_ROWS_BLK, D), lambda i: (i, 0)),
    )(p)


def _tc_finish_body(p_ref, h_ref, g_ref, bt_ref, o_ref):
    t = p_ref[0] + p_ref[1] + h_ref[...]
    mu = jnp.mean(t, axis=-1, keepdims=True)
    var = jnp.mean((t - mu) ** 2, axis=-1, keepdims=True)
    t = (t - mu) * lax.rsqrt(var + 1e-5) * g_ref[...] + bt_ref[...]
    o_ref[...] = t * 0.5 * (1.0 + lax.erf(t * (2.0 ** -0.5)))


def _tc_finish(p, h, gamma, beta):
    return pl.pallas_call(
        _tc_finish_body,
        out_shape=jax.ShapeDtypeStruct((N_NODES, D), jnp.float32),
        grid=(N_NODES // _ROWS_BLK,),
        in_specs=[
            pl.BlockSpec((NC, _ROWS_BLK, D), lambda i: (0, i, 0)),
            pl.BlockSpec((_ROWS_BLK, D), lambda i: (i, 0)),
            pl.BlockSpec((1, D), lambda i: (0, 0)),
            pl.BlockSpec((1, D), lambda i: (0, 0)),
        ],
        out_specs=pl.BlockSpec((_ROWS_BLK, D), lambda i: (i, 0)),
    )(p, h, gamma.reshape(1, D), beta.reshape(1, D))


def _pad_blocks(v, dtype):
    v = v.astype(dtype)
    return jnp.concatenate(
        [v, jnp.zeros((E_PAD - N_EDGES,), dtype)]).reshape(BLKS_PER_ADJ, EPB)


def kernel(x, edge_index_0, edge_weight_0, edge_index_1, edge_weight_1,
           edge_index_2, edge_weight_2, W_aff, b_aff, ln_gamma, ln_beta):
    src01 = jnp.concatenate([_pad_blocks(edge_index_0[0], jnp.int32),
                             _pad_blocks(edge_index_1[0], jnp.int32)])
    dst01 = jnp.concatenate([_pad_blocks(edge_index_0[1], jnp.int32),
                             _pad_blocks(edge_index_1[1], jnp.int32)])
    w01 = jnp.concatenate([_pad_blocks(edge_weight_0, jnp.float32),
                           _pad_blocks(edge_weight_1, jnp.float32)])
    src2 = _pad_blocks(edge_index_2[0], jnp.int32)
    dst2 = _pad_blocks(edge_index_2[1], jnp.int32)
    w2 = _pad_blocks(edge_weight_2, jnp.float32)

    h = _tc_affine(x, W_aff, b_aff)
    p01 = _sc_spmm_pair(src01, dst01, w01, h)
    s1 = _tc_sum_pair(p01)
    p2 = _sc_spmm_single(src2, dst2, w2, s1)
    return _tc_finish(p2, h, ln_gamma, ln_beta)
